# Initial kernel scaffold; baseline (speedup 1.0000x reference)
import jax
import jax.numpy as jnp
from jax.experimental import pallas as pl


def kernel(modality_ids, modality_embedding):
    raise NotImplementedError("write your pallas kernel here")


# speedup vs baseline: 2.7677x; 2.7677x over previous
import functools

import jax
import jax.numpy as jnp
from jax import lax
from jax.experimental import pallas as pl
from jax.experimental.pallas import tpu as pltpu
from jax.experimental.pallas import tpu_sc as plsc

NUM_IDS = 4096 * 200
EMBED = 64
G = 4
ROWW = EMBED * G
NGRP = NUM_IDS // G
NC, NS = 2, 16
NW = NC * NS
PER_W = NGRP // NW
BLK = 128
NB = 2
SQ = BLK * NB
STEPS = PER_W // SQ
L = 16


def _sc_body(planes_hbm, table_hbm, out_hbm, planes_v, idx_v, rows_v, sem):
    wid = lax.axis_index("s") * NC + lax.axis_index("c")
    base_w = wid * PER_W

    def step(i, carry):
        qbase = base_w + i * SQ
        pltpu.sync_copy(planes_hbm.at[pl.ds(qbase // SQ, 1)], planes_v)
        for k in range(SQ // L):
            va = planes_v[0, 0, pl.ds(k * L, L)]
            vb = planes_v[0, 1, pl.ds(k * L, L)]
            vc = planes_v[0, 2, pl.ds(k * L, L)]
            vd = planes_v[0, 3, pl.ds(k * L, L)]
            idx = ((va * 3 + vb) * 3 + vc) * 3 + vd
            idx_v[k // (BLK // L), pl.ds((k % (BLK // L)) * L, L)] = idx
        copies = []
        for j in range(NB):
            copies.append(
                pltpu.async_copy(
                    table_hbm.at[idx_v.at[j]],
                    rows_v.at[pl.ds(j * BLK, BLK)],
                    sem,
                )
            )
        for cp in copies:
            cp.wait()
        pltpu.sync_copy(rows_v, out_hbm.at[pl.ds(qbase, SQ)])
        return carry

    lax.fori_loop(0, STEPS, step, 0)


def kernel(modality_ids, modality_embedding):
    ids = modality_ids.reshape(-1).astype(jnp.int32)
    planes = (
        ids.reshape(NGRP, G).T.reshape(G, NGRP // SQ, SQ).transpose(1, 0, 2)
    )
    t = modality_embedding
    t2 = jnp.concatenate(
        [jnp.repeat(t, 3, axis=0), jnp.tile(t, (3, 1))], axis=1
    )
    t4 = jnp.concatenate(
        [jnp.repeat(t2, 9, axis=0), jnp.tile(t2, (9, 1))], axis=1
    )

    mesh = plsc.VectorSubcoreMesh(core_axis_name="c", subcore_axis_name="s")
    run = functools.partial(
        pl.kernel,
        mesh=mesh,
        out_type=jax.ShapeDtypeStruct((NGRP, ROWW), jnp.float32),
        scratch_types=[
            pltpu.VMEM((1, G, SQ), jnp.int32),
            pltpu.VMEM((NB, BLK), jnp.int32),
            pltpu.VMEM((SQ, ROWW), jnp.float32),
            pltpu.SemaphoreType.DMA,
        ],
    )(_sc_body)
    out = run(planes, t4)
    return out.reshape(modality_ids.shape + (EMBED,))
